# Initial kernel scaffold; baseline (speedup 1.0000x reference)
import jax
import jax.numpy as jnp
from jax.experimental import pallas as pl


def kernel(inputs, e_w, snorm_n, snorm_e, edge_index, W_h, b_h, W_e, b_e, g1_Wz, g1_bz, g1_Wself, g1_bself, g1_Wa, g1_ba, g2_Wz, g2_bz, g2_Wself, g2_bself, g2_Wa, g2_ba, enc_Wih0, enc_Whh0, enc_bih0, enc_bhh0, enc_Wih1, enc_Whh1, enc_bih1, enc_bhh1, dec_Wih0, dec_Whh0, dec_bih0, dec_bhh0, dec_Wih1, dec_Whh1, dec_bih1, dec_bhh1, dec_Wout, dec_bout):
    raise NotImplementedError("write your pallas kernel here")


# speedup vs baseline: 29.0039x; 29.0039x over previous
import functools

import jax
import jax.numpy as jnp
from jax import lax
from jax.experimental import pallas as pl
from jax.experimental.pallas import tpu as pltpu
from jax.experimental.pallas import tpu_sc as plsc

NN = 10000
EE = 160000
TT = 5
IND = 6
HH = 32
GHD = 2
NPRED = 3

NPAD = 10240
DD = 176
NCORES = 2
NSUB = 16
NWORK = NCORES * NSUB
BB = 64
NBATCH = 80
EPW = BB * NBATCH
EPAD = EPW * NWORK
BLK = 512
NBLK = NPAD // BLK

_f32 = jnp.float32


def _sc_edge_body(table_hbm, sdst_hbm, src_hbm, dst_hbm, out_hbm,
                  zs, sdb, srcv, dstv, acc, sem, sem2):
    c = lax.axis_index("c")
    s = lax.axis_index("s")
    lane16 = lax.iota(jnp.int32, 16)
    zeros16 = jnp.zeros((16,), _f32)
    low5 = jnp.where(lane16 < TT, 1.0, 0.0).astype(_f32)

    def _zero_row(i, carry):
        for k in range(DD // 16):
            zs[i, pl.ds(k * 16, 16)] = zeros16
        return carry
    lax.fori_loop(0, BB, _zero_row, 0)
    rows_per_sub = NPAD // NSUB
    for j in range(rows_per_sub // BB):
        pltpu.sync_copy(zs, acc.at[pl.ds(s * rows_per_sub + j * BB, BB)])
    plsc.subcore_barrier()

    wid = c * NSUB + s
    base0 = wid * EPW

    def _batch(b, carry):
        base = base0 + b * BB
        pltpu.sync_copy(src_hbm.at[pl.ds(base, BB)], srcv)
        pltpu.sync_copy(dst_hbm.at[pl.ds(base, BB)], dstv)
        g1 = pltpu.async_copy(table_hbm.at[srcv], zs, sem)
        g2 = pltpu.async_copy(sdst_hbm.at[dstv], sdb, sem2)
        g1.wait()
        g2.wait()

        def _edge(i, carry2):
            srow = zs[i, pl.ds(160, 16)]
            sdrow = sdb[i, pl.ds(0, 16)]
            a = srow + sdrow
            a = jnp.maximum(a, a * 0.2)
            w = jnp.exp(a)
            zs[i, pl.ds(160, 16)] = w * low5
            for t in range(TT):
                wt = lax.broadcast_in_dim(w[t:t + 1], (16,), (0,))
                for k in range(2):
                    col = t * 32 + k * 16
                    zs[i, pl.ds(col, 16)] = zs[i, pl.ds(col, 16)] * wt
            return carry2
        lax.fori_loop(0, BB, _edge, 0)

        pltpu.sync_copy(zs, acc.at[dstv], add=True)
        return carry

    lax.fori_loop(0, NBATCH, _batch, 0)
    plsc.subcore_barrier()

    for j in range(rows_per_sub // BB):
        r0 = s * rows_per_sub + j * BB
        pltpu.sync_copy(acc.at[pl.ds(r0, BB)], zs)
        pltpu.sync_copy(zs, out_hbm.at[c, pl.ds(r0, BB)])


_sc_edge_pass = functools.partial(
    pl.kernel,
    out_type=jax.ShapeDtypeStruct((NCORES, NPAD, DD), _f32),
    mesh=plsc.VectorSubcoreMesh(core_axis_name="c", subcore_axis_name="s",
                                num_cores=NCORES, num_subcores=NSUB),
    scratch_types=[
        pltpu.VMEM((BB, DD), _f32),
        pltpu.VMEM((BB, 16), _f32),
        pltpu.VMEM((BB,), jnp.int32),
        pltpu.VMEM((BB,), jnp.int32),
        pltpu.VMEM_SHARED((NPAD, DD), _f32),
        pltpu.SemaphoreType.DMA,
        pltpu.SemaphoreType.DMA,
    ],
    compiler_params=pltpu.CompilerParams(use_tc_tiling_on_sc=False),
)(_sc_edge_body)


def _dense_stage(h_t, Wz, bz, Ws, bs, Wa2, ba):
    z_t = jnp.dot(h_t, Wz, preferred_element_type=_f32) + bz
    self_t = jnp.dot(h_t, Ws, preferred_element_type=_f32) + bs
    sc = jnp.dot(z_t, Wa2, preferred_element_type=_f32)
    return z_t, self_t, sc[:, 0:1], sc[:, 1:2] + ba


def _tc1_body(x_ref, Wh_ref, bh_ref, Wz_ref, bz_ref, Ws_ref, bs_ref,
              Wa_ref, ba_ref, table_ref, self_ref, sdst_ref):
    x = x_ref[...]
    ssrc_cols = []
    sdst_cols = []
    for t in range(TT):
        h_t = jnp.dot(x[:, t * IND:(t + 1) * IND], Wh_ref[...],
                      preferred_element_type=_f32) + bh_ref[...]
        z_t, self_t, ssrc, sdst = _dense_stage(
            h_t, Wz_ref[...], bz_ref[...], Ws_ref[...], bs_ref[...],
            Wa_ref[...], ba_ref[0, 0])
        table_ref[:, t * HH:(t + 1) * HH] = z_t
        self_ref[:, t * HH:(t + 1) * HH] = self_t
        ssrc_cols.append(ssrc)
        sdst_cols.append(sdst)
    pad = jnp.zeros((x.shape[0], 16 - TT), _f32)
    table_ref[:, 160:176] = jnp.concatenate(ssrc_cols + [pad], axis=1)
    sdst_ref[...] = jnp.concatenate(
        sdst_cols + [jnp.zeros((x.shape[0], 16 - TT), _f32)], axis=1)


def _finish_gat(part, self_ref, snorm, t):
    numer = part[0, :, t * HH:(t + 1) * HH] + part[1, :, t * HH:(t + 1) * HH]
    den = part[0, :, 160 + t:161 + t] + part[1, :, 160 + t:161 + t]
    h_agg = numer / (den + 1e-9)
    return jax.nn.relu(h_agg * snorm + self_ref[:, t * HH:(t + 1) * HH])


def _tc2_body(part_ref, self1_ref, snorm_ref, Wz_ref, bz_ref, Ws_ref, bs_ref,
              Wa_ref, ba_ref, table_ref, self_ref, sdst_ref):
    part = part_ref[...]
    snorm = snorm_ref[...]
    ssrc_cols = []
    sdst_cols = []
    for t in range(TT):
        h_t = _finish_gat(part, self1_ref, snorm, t)
        z_t, self_t, ssrc, sdst = _dense_stage(
            h_t, Wz_ref[...], bz_ref[...], Ws_ref[...], bs_ref[...],
            Wa_ref[...], ba_ref[0, 0])
        table_ref[:, t * HH:(t + 1) * HH] = z_t
        self_ref[:, t * HH:(t + 1) * HH] = self_t
        ssrc_cols.append(ssrc)
        sdst_cols.append(sdst)
    nrow = part.shape[1]
    table_ref[:, 160:176] = jnp.concatenate(
        ssrc_cols + [jnp.zeros((nrow, 16 - TT), _f32)], axis=1)
    sdst_ref[...] = jnp.concatenate(
        sdst_cols + [jnp.zeros((nrow, 16 - TT), _f32)], axis=1)


def _mat2(x, W, b):
    return x[:, 0:1] * W[0:1, :] + x[:, 1:2] * W[1:2, :] + b


def _gru_step(gi, gh, h):
    g = GHD
    r = jax.nn.sigmoid(gi[:, :g] + gh[:, :g])
    z = jax.nn.sigmoid(gi[:, g:2 * g] + gh[:, g:2 * g])
    n = jnp.tanh(gi[:, 2 * g:] + r * gh[:, 2 * g:])
    return (1.0 - z) * n + z * h


def _tc3_body(part_ref, self2_ref, snorm_ref, x_ref,
              eW0_ref, eU0_ref, eb0_ref, ec0_ref,
              eW1_ref, eU1_ref, eb1_ref, ec1_ref,
              dW0_ref, dU0_ref, db0_ref, dc0_ref,
              dW1_ref, dU1_ref, db1_ref, dc1_ref,
              Wo_ref, bo_ref, out_ref):
    part = part_ref[...]
    snorm = snorm_ref[...]
    nrow = part.shape[1]
    h0 = jnp.zeros((nrow, GHD), _f32)
    h1 = jnp.zeros((nrow, GHD), _f32)
    for t in range(TT):
        x_t = _finish_gat(part, self2_ref, snorm, t)
        gi0 = jnp.dot(x_t, eW0_ref[...], preferred_element_type=_f32) \
            + eb0_ref[...]
        h0 = _gru_step(gi0, _mat2(h0, eU0_ref[...], ec0_ref[...]), h0)
        gi1 = _mat2(h0, eW1_ref[...], eb1_ref[...])
        h1 = _gru_step(gi1, _mat2(h1, eU1_ref[...], ec1_ref[...]), h1)
    dec_in = x_ref[:, (TT - 1) * IND:(TT - 1) * IND + 2]
    for p in range(NPRED):
        gi0 = _mat2(dec_in, dW0_ref[...], db0_ref[...])
        h0 = _gru_step(gi0, _mat2(h0, dU0_ref[...], dc0_ref[...]), h0)
        gi1 = _mat2(h0, dW1_ref[...], db1_ref[...])
        h1 = _gru_step(gi1, _mat2(h1, dU1_ref[...], dc1_ref[...]), h1)
        out = _mat2(h1, Wo_ref[...], bo_ref[...]) + dec_in
        out_ref[:, p * 2:(p + 1) * 2] = out
        dec_in = out


def _full_spec(shape):
    return pl.BlockSpec(shape, lambda i: tuple(0 for _ in shape))


def _row_spec(cols):
    return pl.BlockSpec((BLK, cols), lambda i: (i, 0))


def kernel(inputs, e_w, snorm_n, snorm_e, edge_index, W_h, b_h, W_e, b_e,
           g1_Wz, g1_bz, g1_Wself, g1_bself, g1_Wa, g1_ba,
           g2_Wz, g2_bz, g2_Wself, g2_bself, g2_Wa, g2_ba,
           enc_Wih0, enc_Whh0, enc_bih0, enc_bhh0,
           enc_Wih1, enc_Whh1, enc_bih1, enc_bhh1,
           dec_Wih0, dec_Whh0, dec_bih0, dec_bhh0,
           dec_Wih1, dec_Whh1, dec_bih1, dec_bhh1,
           dec_Wout, dec_bout):
    x2d = jnp.pad(inputs.reshape(NN, TT * IND), ((0, NPAD - NN), (0, 0)))
    snorm = jnp.pad(snorm_n.reshape(NN, 1), ((0, NPAD - NN), (0, 0)))
    src = jnp.pad(edge_index[0], (0, EPAD - EE))
    dst = jnp.pad(edge_index[1], (0, EPAD - EE), constant_values=NN)
    bh = b_h.reshape(1, HH)
    Wa1 = jnp.concatenate([g1_Wa[:HH], g1_Wa[HH:]], axis=1)
    Wa2 = jnp.concatenate([g2_Wa[:HH], g2_Wa[HH:]], axis=1)
    ba1 = g1_ba.reshape(1, 1)
    ba2 = g2_ba.reshape(1, 1)

    w_specs = [_full_spec(s) for s in
               ((IND, HH), (1, HH), (HH, HH), (1, HH), (HH, HH), (1, HH),
                (HH, 2), (1, 1))]

    table1, self1, sdst1 = pl.pallas_call(
        _tc1_body,
        grid=(NBLK,),
        in_specs=[_row_spec(TT * IND)] + w_specs,
        out_specs=[_row_spec(DD), _row_spec(160), _row_spec(16)],
        out_shape=[jax.ShapeDtypeStruct((NPAD, DD), _f32),
                   jax.ShapeDtypeStruct((NPAD, 160), _f32),
                   jax.ShapeDtypeStruct((NPAD, 16), _f32)],
    )(x2d, W_h, bh, g1_Wz, g1_bz.reshape(1, HH), g1_Wself,
      g1_bself.reshape(1, HH), Wa1, ba1)

    part1 = _sc_edge_pass(table1, sdst1, src, dst)

    part_spec = pl.BlockSpec((NCORES, BLK, DD), lambda i: (0, i, 0))
    table2, self2, sdst2 = pl.pallas_call(
        _tc2_body,
        grid=(NBLK,),
        in_specs=[part_spec, _row_spec(160), _row_spec(1)] + w_specs[2:],
        out_specs=[_row_spec(DD), _row_spec(160), _row_spec(16)],
        out_shape=[jax.ShapeDtypeStruct((NPAD, DD), _f32),
                   jax.ShapeDtypeStruct((NPAD, 160), _f32),
                   jax.ShapeDtypeStruct((NPAD, 16), _f32)],
    )(part1, self1, snorm, g2_Wz, g2_bz.reshape(1, HH), g2_Wself,
      g2_bself.reshape(1, HH), Wa2, ba2)

    part2 = _sc_edge_pass(table2, sdst2, src, dst)

    g3 = 3 * GHD
    gru_specs = [_full_spec(s) for s in
                 ((HH, g3), (GHD, g3), (1, g3), (1, g3),
                  (GHD, g3), (GHD, g3), (1, g3), (1, g3),
                  (2, g3), (GHD, g3), (1, g3), (1, g3),
                  (GHD, g3), (GHD, g3), (1, g3), (1, g3),
                  (GHD, 2), (1, 2))]

    out_p = pl.pallas_call(
        _tc3_body,
        grid=(NBLK,),
        in_specs=[part_spec, _row_spec(160), _row_spec(1),
                  _row_spec(TT * IND)] + gru_specs,
        out_specs=_row_spec(2 * NPRED),
        out_shape=jax.ShapeDtypeStruct((NPAD, 2 * NPRED), _f32),
    )(part2, self2, snorm, x2d,
      enc_Wih0, enc_Whh0, enc_bih0.reshape(1, g3), enc_bhh0.reshape(1, g3),
      enc_Wih1, enc_Whh1, enc_bih1.reshape(1, g3), enc_bhh1.reshape(1, g3),
      dec_Wih0, dec_Whh0, dec_bih0.reshape(1, g3), dec_bhh0.reshape(1, g3),
      dec_Wih1, dec_Whh1, dec_bih1.reshape(1, g3), dec_bhh1.reshape(1, g3),
      dec_Wout, dec_bout.reshape(1, 2))

    return out_p[:NN].reshape(NN, NPRED, 2)


# speedup vs baseline: 32.0671x; 1.1056x over previous
import functools

import jax
import jax.numpy as jnp
from jax import lax
from jax.experimental import pallas as pl
from jax.experimental.pallas import tpu as pltpu
from jax.experimental.pallas import tpu_sc as plsc

NN = 10000
EE = 160000
TT = 5
IND = 6
HH = 32
GHD = 2
NPRED = 3

NPAD = 10240
DD = 176
NCORES = 2
NSUB = 16
NWORK = NCORES * NSUB
BB = 64
NBATCH = 80
EPW = BB * NBATCH
EPAD = EPW * NWORK
BLK = 512
NBLK = NPAD // BLK

_f32 = jnp.float32


def _sc_edge_body(table_hbm, sdst_hbm, src_hbm, dst_hbm, out_hbm,
                  zs, sdb, srcv, dstv, acc, sem, sem2):
    c = lax.axis_index("c")
    s = lax.axis_index("s")
    lane16 = lax.iota(jnp.int32, 16)
    zeros16 = jnp.zeros((16,), _f32)
    low5 = jnp.where(lane16 < TT, 1.0, 0.0).astype(_f32)

    def _zero_row(i, carry):
        for k in range(DD // 16):
            zs[i, pl.ds(k * 16, 16)] = zeros16
        return carry
    lax.fori_loop(0, BB, _zero_row, 0)
    rows_per_sub = NPAD // NSUB
    for j in range(rows_per_sub // BB):
        pltpu.sync_copy(zs, acc.at[pl.ds(s * rows_per_sub + j * BB, BB)])
    plsc.subcore_barrier()

    wid = c * NSUB + s
    base0 = wid * EPW

    def _batch(b, carry):
        base = base0 + b * BB
        pltpu.sync_copy(src_hbm.at[pl.ds(base, BB)], srcv)
        pltpu.sync_copy(dst_hbm.at[pl.ds(base, BB)], dstv)
        g1 = pltpu.async_copy(table_hbm.at[srcv], zs, sem)
        g2 = pltpu.async_copy(sdst_hbm.at[dstv], sdb, sem2)
        g1.wait()
        g2.wait()

        @plsc.parallel_loop(0, BB, step=1, unroll=4)
        def _edge(i):
            srow = zs[i, pl.ds(160, 16)]
            sdrow = sdb[i, pl.ds(0, 16)]
            a = srow + sdrow
            a = jnp.maximum(a, a * 0.2)
            w = jnp.exp(a)
            zs[i, pl.ds(160, 16)] = w * low5
            for t in range(TT):
                wt = lax.broadcast_in_dim(w[t:t + 1], (16,), (0,))
                for k in range(2):
                    col = t * 32 + k * 16
                    zs[i, pl.ds(col, 16)] = zs[i, pl.ds(col, 16)] * wt

        pltpu.sync_copy(zs, acc.at[dstv], add=True)
        return carry

    lax.fori_loop(0, NBATCH, _batch, 0)
    plsc.subcore_barrier()

    for j in range(rows_per_sub // BB):
        r0 = s * rows_per_sub + j * BB
        pltpu.sync_copy(acc.at[pl.ds(r0, BB)], zs)
        pltpu.sync_copy(zs, out_hbm.at[c, pl.ds(r0, BB)])


_sc_edge_pass = functools.partial(
    pl.kernel,
    out_type=jax.ShapeDtypeStruct((NCORES, NPAD, DD), _f32),
    mesh=plsc.VectorSubcoreMesh(core_axis_name="c", subcore_axis_name="s",
                                num_cores=NCORES, num_subcores=NSUB),
    scratch_types=[
        pltpu.VMEM((BB, DD), _f32),
        pltpu.VMEM((BB, 16), _f32),
        pltpu.VMEM((BB,), jnp.int32),
        pltpu.VMEM((BB,), jnp.int32),
        pltpu.VMEM_SHARED((NPAD, DD), _f32),
        pltpu.SemaphoreType.DMA,
        pltpu.SemaphoreType.DMA,
    ],
    compiler_params=pltpu.CompilerParams(use_tc_tiling_on_sc=False),
)(_sc_edge_body)


def _dense_stage(h_t, Wz, bz, Ws, bs, Wa2, ba):
    z_t = jnp.dot(h_t, Wz, preferred_element_type=_f32) + bz
    self_t = jnp.dot(h_t, Ws, preferred_element_type=_f32) + bs
    sc = jnp.dot(z_t, Wa2, preferred_element_type=_f32)
    return z_t, self_t, sc[:, 0:1], sc[:, 1:2] + ba


def _tc1_body(x_ref, Wh_ref, bh_ref, Wz_ref, bz_ref, Ws_ref, bs_ref,
              Wa_ref, ba_ref, table_ref, self_ref, sdst_ref):
    x = x_ref[...]
    ssrc_cols = []
    sdst_cols = []
    for t in range(TT):
        h_t = jnp.dot(x[:, t * IND:(t + 1) * IND], Wh_ref[...],
                      preferred_element_type=_f32) + bh_ref[...]
        z_t, self_t, ssrc, sdst = _dense_stage(
            h_t, Wz_ref[...], bz_ref[...], Ws_ref[...], bs_ref[...],
            Wa_ref[...], ba_ref[0, 0])
        table_ref[:, t * HH:(t + 1) * HH] = z_t
        self_ref[:, t * HH:(t + 1) * HH] = self_t
        ssrc_cols.append(ssrc)
        sdst_cols.append(sdst)
    pad = jnp.zeros((x.shape[0], 16 - TT), _f32)
    table_ref[:, 160:176] = jnp.concatenate(ssrc_cols + [pad], axis=1)
    sdst_ref[...] = jnp.concatenate(
        sdst_cols + [jnp.zeros((x.shape[0], 16 - TT), _f32)], axis=1)


def _finish_gat(part, self_ref, snorm, t):
    numer = part[0, :, t * HH:(t + 1) * HH] + part[1, :, t * HH:(t + 1) * HH]
    den = part[0, :, 160 + t:161 + t] + part[1, :, 160 + t:161 + t]
    h_agg = numer / (den + 1e-9)
    return jax.nn.relu(h_agg * snorm + self_ref[:, t * HH:(t + 1) * HH])


def _tc2_body(part_ref, self1_ref, snorm_ref, Wz_ref, bz_ref, Ws_ref, bs_ref,
              Wa_ref, ba_ref, table_ref, self_ref, sdst_ref):
    part = part_ref[...]
    snorm = snorm_ref[...]
    ssrc_cols = []
    sdst_cols = []
    for t in range(TT):
        h_t = _finish_gat(part, self1_ref, snorm, t)
        z_t, self_t, ssrc, sdst = _dense_stage(
            h_t, Wz_ref[...], bz_ref[...], Ws_ref[...], bs_ref[...],
            Wa_ref[...], ba_ref[0, 0])
        table_ref[:, t * HH:(t + 1) * HH] = z_t
        self_ref[:, t * HH:(t + 1) * HH] = self_t
        ssrc_cols.append(ssrc)
        sdst_cols.append(sdst)
    nrow = part.shape[1]
    table_ref[:, 160:176] = jnp.concatenate(
        ssrc_cols + [jnp.zeros((nrow, 16 - TT), _f32)], axis=1)
    sdst_ref[...] = jnp.concatenate(
        sdst_cols + [jnp.zeros((nrow, 16 - TT), _f32)], axis=1)


def _mat2(x, W, b):
    return x[:, 0:1] * W[0:1, :] + x[:, 1:2] * W[1:2, :] + b


def _gru_step(gi, gh, h):
    g = GHD
    r = jax.nn.sigmoid(gi[:, :g] + gh[:, :g])
    z = jax.nn.sigmoid(gi[:, g:2 * g] + gh[:, g:2 * g])
    n = jnp.tanh(gi[:, 2 * g:] + r * gh[:, 2 * g:])
    return (1.0 - z) * n + z * h


def _tc3_body(part_ref, self2_ref, snorm_ref, x_ref,
              eW0_ref, eU0_ref, eb0_ref, ec0_ref,
              eW1_ref, eU1_ref, eb1_ref, ec1_ref,
              dW0_ref, dU0_ref, db0_ref, dc0_ref,
              dW1_ref, dU1_ref, db1_ref, dc1_ref,
              Wo_ref, bo_ref, out_ref):
    part = part_ref[...]
    snorm = snorm_ref[...]
    nrow = part.shape[1]
    h0 = jnp.zeros((nrow, GHD), _f32)
    h1 = jnp.zeros((nrow, GHD), _f32)
    for t in range(TT):
        x_t = _finish_gat(part, self2_ref, snorm, t)
        gi0 = jnp.dot(x_t, eW0_ref[...], preferred_element_type=_f32) \
            + eb0_ref[...]
        h0 = _gru_step(gi0, _mat2(h0, eU0_ref[...], ec0_ref[...]), h0)
        gi1 = _mat2(h0, eW1_ref[...], eb1_ref[...])
        h1 = _gru_step(gi1, _mat2(h1, eU1_ref[...], ec1_ref[...]), h1)
    dec_in = x_ref[:, (TT - 1) * IND:(TT - 1) * IND + 2]
    for p in range(NPRED):
        gi0 = _mat2(dec_in, dW0_ref[...], db0_ref[...])
        h0 = _gru_step(gi0, _mat2(h0, dU0_ref[...], dc0_ref[...]), h0)
        gi1 = _mat2(h0, dW1_ref[...], db1_ref[...])
        h1 = _gru_step(gi1, _mat2(h1, dU1_ref[...], dc1_ref[...]), h1)
        out = _mat2(h1, Wo_ref[...], bo_ref[...]) + dec_in
        out_ref[:, p * 2:(p + 1) * 2] = out
        dec_in = out


def _full_spec(shape):
    return pl.BlockSpec(shape, lambda i: tuple(0 for _ in shape))


def _row_spec(cols):
    return pl.BlockSpec((BLK, cols), lambda i: (i, 0))


def kernel(inputs, e_w, snorm_n, snorm_e, edge_index, W_h, b_h, W_e, b_e,
           g1_Wz, g1_bz, g1_Wself, g1_bself, g1_Wa, g1_ba,
           g2_Wz, g2_bz, g2_Wself, g2_bself, g2_Wa, g2_ba,
           enc_Wih0, enc_Whh0, enc_bih0, enc_bhh0,
           enc_Wih1, enc_Whh1, enc_bih1, enc_bhh1,
           dec_Wih0, dec_Whh0, dec_bih0, dec_bhh0,
           dec_Wih1, dec_Whh1, dec_bih1, dec_bhh1,
           dec_Wout, dec_bout):
    x2d = jnp.pad(inputs.reshape(NN, TT * IND), ((0, NPAD - NN), (0, 0)))
    snorm = jnp.pad(snorm_n.reshape(NN, 1), ((0, NPAD - NN), (0, 0)))
    src = jnp.pad(edge_index[0], (0, EPAD - EE))
    dst = jnp.pad(edge_index[1], (0, EPAD - EE), constant_values=NN)
    bh = b_h.reshape(1, HH)
    Wa1 = jnp.concatenate([g1_Wa[:HH], g1_Wa[HH:]], axis=1)
    Wa2 = jnp.concatenate([g2_Wa[:HH], g2_Wa[HH:]], axis=1)
    ba1 = g1_ba.reshape(1, 1)
    ba2 = g2_ba.reshape(1, 1)

    w_specs = [_full_spec(s) for s in
               ((IND, HH), (1, HH), (HH, HH), (1, HH), (HH, HH), (1, HH),
                (HH, 2), (1, 1))]

    table1, self1, sdst1 = pl.pallas_call(
        _tc1_body,
        grid=(NBLK,),
        in_specs=[_row_spec(TT * IND)] + w_specs,
        out_specs=[_row_spec(DD), _row_spec(160), _row_spec(16)],
        out_shape=[jax.ShapeDtypeStruct((NPAD, DD), _f32),
                   jax.ShapeDtypeStruct((NPAD, 160), _f32),
                   jax.ShapeDtypeStruct((NPAD, 16), _f32)],
    )(x2d, W_h, bh, g1_Wz, g1_bz.reshape(1, HH), g1_Wself,
      g1_bself.reshape(1, HH), Wa1, ba1)

    part1 = _sc_edge_pass(table1, sdst1, src, dst)

    part_spec = pl.BlockSpec((NCORES, BLK, DD), lambda i: (0, i, 0))
    table2, self2, sdst2 = pl.pallas_call(
        _tc2_body,
        grid=(NBLK,),
        in_specs=[part_spec, _row_spec(160), _row_spec(1)] + w_specs[2:],
        out_specs=[_row_spec(DD), _row_spec(160), _row_spec(16)],
        out_shape=[jax.ShapeDtypeStruct((NPAD, DD), _f32),
                   jax.ShapeDtypeStruct((NPAD, 160), _f32),
                   jax.ShapeDtypeStruct((NPAD, 16), _f32)],
    )(part1, self1, snorm, g2_Wz, g2_bz.reshape(1, HH), g2_Wself,
      g2_bself.reshape(1, HH), Wa2, ba2)

    part2 = _sc_edge_pass(table2, sdst2, src, dst)

    g3 = 3 * GHD
    gru_specs = [_full_spec(s) for s in
                 ((HH, g3), (GHD, g3), (1, g3), (1, g3),
                  (GHD, g3), (GHD, g3), (1, g3), (1, g3),
                  (2, g3), (GHD, g3), (1, g3), (1, g3),
                  (GHD, g3), (GHD, g3), (1, g3), (1, g3),
                  (GHD, 2), (1, 2))]

    out_p = pl.pallas_call(
        _tc3_body,
        grid=(NBLK,),
        in_specs=[part_spec, _row_spec(160), _row_spec(1),
                  _row_spec(TT * IND)] + gru_specs,
        out_specs=_row_spec(2 * NPRED),
        out_shape=jax.ShapeDtypeStruct((NPAD, 2 * NPRED), _f32),
    )(part2, self2, snorm, x2d,
      enc_Wih0, enc_Whh0, enc_bih0.reshape(1, g3), enc_bhh0.reshape(1, g3),
      enc_Wih1, enc_Whh1, enc_bih1.reshape(1, g3), enc_bhh1.reshape(1, g3),
      dec_Wih0, dec_Whh0, dec_bih0.reshape(1, g3), dec_bhh0.reshape(1, g3),
      dec_Wih1, dec_Whh1, dec_bih1.reshape(1, g3), dec_bhh1.reshape(1, g3),
      dec_Wout, dec_bout.reshape(1, 2))

    return out_p[:NN].reshape(NN, NPRED, 2)


# speedup vs baseline: 39.3272x; 1.2264x over previous
import functools

import jax
import jax.numpy as jnp
from jax import lax
from jax.experimental import pallas as pl
from jax.experimental.pallas import tpu as pltpu
from jax.experimental.pallas import tpu_sc as plsc

NN = 10000
EE = 160000
TT = 5
IND = 6
HH = 32
GHD = 2
NPRED = 3

NPAD = 10240
DD = 176
NCORES = 2
NSUB = 16
NWORK = NCORES * NSUB
BB = 32
NBATCH = 160
CHK = 16
NCHUNK = NBATCH // CHK
EPW = BB * NBATCH
EPAD = EPW * NWORK
BLK = 512
NBLK = NPAD // BLK

_f32 = jnp.float32


def _sc_edge_body(table_hbm, sdst_hbm, src_hbm, dst_hbm, out_hbm,
                  zs0, zs1, sdb0, sdb1, srcc, dstc, acc,
                  sem_z0, sem_z1, sem_s0, sem_s1):
    c = lax.axis_index("c")
    s = lax.axis_index("s")
    lane16 = lax.iota(jnp.int32, 16)
    zeros16 = jnp.zeros((16,), _f32)
    low5 = jnp.where(lane16 < TT, 1.0, 0.0).astype(_f32)
    zbufs = (zs0, zs1)
    sbufs = (sdb0, sdb1)
    zsems = (sem_z0, sem_z1)
    ssems = (sem_s0, sem_s1)

    def _zero_row(i, carry):
        for k in range(DD // 16):
            zs0[i, pl.ds(k * 16, 16)] = zeros16
        return carry
    lax.fori_loop(0, BB, _zero_row, 0)
    rows_per_sub = NPAD // NSUB
    for j in range(rows_per_sub // BB):
        pltpu.sync_copy(zs0, acc.at[pl.ds(s * rows_per_sub + j * BB, BB)])
    plsc.subcore_barrier()

    wid = c * NSUB + s

    def _weight_rows(zs, sdb):
        @plsc.parallel_loop(0, BB, step=1, unroll=4)
        def _edge(i):
            srow = zs[i, pl.ds(160, 16)]
            sdrow = sdb[i, pl.ds(0, 16)]
            a = srow + sdrow
            a = jnp.maximum(a, a * 0.2)
            w = jnp.exp(a)
            zs[i, pl.ds(160, 16)] = w * low5
            for t in range(TT):
                wt = lax.broadcast_in_dim(w[t:t + 1], (16,), (0,))
                for k in range(2):
                    col = t * 32 + k * 16
                    zs[i, pl.ds(col, 16)] = zs[i, pl.ds(col, 16)] * wt

    def _gathers(j, buf):
        gz = pltpu.async_copy(
            table_hbm.at[srcc.at[pl.ds(j * BB, BB)]], zbufs[buf], zsems[buf])
        gs = pltpu.async_copy(
            sdst_hbm.at[dstc.at[j]], sbufs[buf], ssems[buf])
        return gz, gs

    def _chunk(ch, carry):
        pltpu.sync_copy(src_hbm.at[wid, ch], srcc)
        pltpu.sync_copy(dst_hbm.at[wid, ch], dstc)
        pend = [_gathers(0, 0), _gathers(1, 1)]
        for j in range(CHK):
            buf = j % 2
            gz, gs = pend[buf]
            gz.wait()
            gs.wait()
            _weight_rows(zbufs[buf], sbufs[buf])
            pltpu.sync_copy(zbufs[buf], acc.at[dstc.at[j]], add=True)
            if j + 2 < CHK:
                pend[buf] = _gathers(j + 2, buf)
        return carry

    lax.fori_loop(0, NCHUNK, _chunk, 0)
    plsc.subcore_barrier()

    for j in range(rows_per_sub // BB):
        r0 = s * rows_per_sub + j * BB
        pltpu.sync_copy(acc.at[pl.ds(r0, BB)], zs0)
        pltpu.sync_copy(zs0, out_hbm.at[c, pl.ds(r0, BB)])


_sc_edge_pass = functools.partial(
    pl.kernel,
    out_type=jax.ShapeDtypeStruct((NCORES, NPAD, DD), _f32),
    mesh=plsc.VectorSubcoreMesh(core_axis_name="c", subcore_axis_name="s",
                                num_cores=NCORES, num_subcores=NSUB),
    scratch_types=[
        pltpu.VMEM((BB, DD), _f32),
        pltpu.VMEM((BB, DD), _f32),
        pltpu.VMEM((BB, 16), _f32),
        pltpu.VMEM((BB, 16), _f32),
        pltpu.VMEM((CHK * BB,), jnp.int32),
        pltpu.VMEM((CHK, BB), jnp.int32),
        pltpu.VMEM_SHARED((NPAD, DD), _f32),
        pltpu.SemaphoreType.DMA,
        pltpu.SemaphoreType.DMA,
        pltpu.SemaphoreType.DMA,
        pltpu.SemaphoreType.DMA,
    ],
    compiler_params=pltpu.CompilerParams(use_tc_tiling_on_sc=False),
)(_sc_edge_body)


def _dense_stage(h_t, Wz, bz, Ws, bs, Wa2, ba):
    z_t = jnp.dot(h_t, Wz, preferred_element_type=_f32) + bz
    self_t = jnp.dot(h_t, Ws, preferred_element_type=_f32) + bs
    sc = jnp.dot(z_t, Wa2, preferred_element_type=_f32)
    return z_t, self_t, sc[:, 0:1], sc[:, 1:2] + ba


def _tc1_body(x_ref, Wh_ref, bh_ref, Wz_ref, bz_ref, Ws_ref, bs_ref,
              Wa_ref, ba_ref, table_ref, self_ref, sdst_ref):
    x = x_ref[...]
    ssrc_cols = []
    sdst_cols = []
    for t in range(TT):
        h_t = jnp.dot(x[:, t * IND:(t + 1) * IND], Wh_ref[...],
                      preferred_element_type=_f32) + bh_ref[...]
        z_t, self_t, ssrc, sdst = _dense_stage(
            h_t, Wz_ref[...], bz_ref[...], Ws_ref[...], bs_ref[...],
            Wa_ref[...], ba_ref[0, 0])
        table_ref[:, t * HH:(t + 1) * HH] = z_t
        self_ref[:, t * HH:(t + 1) * HH] = self_t
        ssrc_cols.append(ssrc)
        sdst_cols.append(sdst)
    pad = jnp.zeros((x.shape[0], 16 - TT), _f32)
    table_ref[:, 160:176] = jnp.concatenate(ssrc_cols + [pad], axis=1)
    sdst_ref[...] = jnp.concatenate(
        sdst_cols + [jnp.zeros((x.shape[0], 16 - TT), _f32)], axis=1)


def _finish_gat(part, self_ref, snorm, t):
    numer = part[0, :, t * HH:(t + 1) * HH] + part[1, :, t * HH:(t + 1) * HH]
    den = part[0, :, 160 + t:161 + t] + part[1, :, 160 + t:161 + t]
    h_agg = numer / (den + 1e-9)
    return jax.nn.relu(h_agg * snorm + self_ref[:, t * HH:(t + 1) * HH])


def _tc2_body(part_ref, self1_ref, snorm_ref, Wz_ref, bz_ref, Ws_ref, bs_ref,
              Wa_ref, ba_ref, table_ref, self_ref, sdst_ref):
    part = part_ref[...]
    snorm = snorm_ref[...]
    ssrc_cols = []
    sdst_cols = []
    for t in range(TT):
        h_t = _finish_gat(part, self1_ref, snorm, t)
        z_t, self_t, ssrc, sdst = _dense_stage(
            h_t, Wz_ref[...], bz_ref[...], Ws_ref[...], bs_ref[...],
            Wa_ref[...], ba_ref[0, 0])
        table_ref[:, t * HH:(t + 1) * HH] = z_t
        self_ref[:, t * HH:(t + 1) * HH] = self_t
        ssrc_cols.append(ssrc)
        sdst_cols.append(sdst)
    nrow = part.shape[1]
    table_ref[:, 160:176] = jnp.concatenate(
        ssrc_cols + [jnp.zeros((nrow, 16 - TT), _f32)], axis=1)
    sdst_ref[...] = jnp.concatenate(
        sdst_cols + [jnp.zeros((nrow, 16 - TT), _f32)], axis=1)


def _mat2(x, W, b):
    return x[:, 0:1] * W[0:1, :] + x[:, 1:2] * W[1:2, :] + b


def _gru_step(gi, gh, h):
    g = GHD
    r = jax.nn.sigmoid(gi[:, :g] + gh[:, :g])
    z = jax.nn.sigmoid(gi[:, g:2 * g] + gh[:, g:2 * g])
    n = jnp.tanh(gi[:, 2 * g:] + r * gh[:, 2 * g:])
    return (1.0 - z) * n + z * h


def _tc3_body(part_ref, self2_ref, snorm_ref, x_ref,
              eW0_ref, eU0_ref, eb0_ref, ec0_ref,
              eW1_ref, eU1_ref, eb1_ref, ec1_ref,
              dW0_ref, dU0_ref, db0_ref, dc0_ref,
              dW1_ref, dU1_ref, db1_ref, dc1_ref,
              Wo_ref, bo_ref, out_ref):
    part = part_ref[...]
    snorm = snorm_ref[...]
    nrow = part.shape[1]
    h0 = jnp.zeros((nrow, GHD), _f32)
    h1 = jnp.zeros((nrow, GHD), _f32)
    for t in range(TT):
        x_t = _finish_gat(part, self2_ref, snorm, t)
        gi0 = jnp.dot(x_t, eW0_ref[...], preferred_element_type=_f32) \
            + eb0_ref[...]
        h0 = _gru_step(gi0, _mat2(h0, eU0_ref[...], ec0_ref[...]), h0)
        gi1 = _mat2(h0, eW1_ref[...], eb1_ref[...])
        h1 = _gru_step(gi1, _mat2(h1, eU1_ref[...], ec1_ref[...]), h1)
    dec_in = x_ref[:, (TT - 1) * IND:(TT - 1) * IND + 2]
    for p in range(NPRED):
        gi0 = _mat2(dec_in, dW0_ref[...], db0_ref[...])
        h0 = _gru_step(gi0, _mat2(h0, dU0_ref[...], dc0_ref[...]), h0)
        gi1 = _mat2(h0, dW1_ref[...], db1_ref[...])
        h1 = _gru_step(gi1, _mat2(h1, dU1_ref[...], dc1_ref[...]), h1)
        out = _mat2(h1, Wo_ref[...], bo_ref[...]) + dec_in
        out_ref[:, p * 2:(p + 1) * 2] = out
        dec_in = out


def _full_spec(shape):
    return pl.BlockSpec(shape, lambda i: tuple(0 for _ in shape))


def _row_spec(cols):
    return pl.BlockSpec((BLK, cols), lambda i: (i, 0))


def kernel(inputs, e_w, snorm_n, snorm_e, edge_index, W_h, b_h, W_e, b_e,
           g1_Wz, g1_bz, g1_Wself, g1_bself, g1_Wa, g1_ba,
           g2_Wz, g2_bz, g2_Wself, g2_bself, g2_Wa, g2_ba,
           enc_Wih0, enc_Whh0, enc_bih0, enc_bhh0,
           enc_Wih1, enc_Whh1, enc_bih1, enc_bhh1,
           dec_Wih0, dec_Whh0, dec_bih0, dec_bhh0,
           dec_Wih1, dec_Whh1, dec_bih1, dec_bhh1,
           dec_Wout, dec_bout):
    x2d = jnp.pad(inputs.reshape(NN, TT * IND), ((0, NPAD - NN), (0, 0)))
    snorm = jnp.pad(snorm_n.reshape(NN, 1), ((0, NPAD - NN), (0, 0)))
    src = jnp.pad(edge_index[0], (0, EPAD - EE)).reshape(
        NWORK, NCHUNK, CHK * BB)
    dst = jnp.pad(edge_index[1], (0, EPAD - EE), constant_values=NN).reshape(
        NWORK, NCHUNK, CHK, BB)
    bh = b_h.reshape(1, HH)
    Wa1 = jnp.concatenate([g1_Wa[:HH], g1_Wa[HH:]], axis=1)
    Wa2 = jnp.concatenate([g2_Wa[:HH], g2_Wa[HH:]], axis=1)
    ba1 = g1_ba.reshape(1, 1)
    ba2 = g2_ba.reshape(1, 1)

    w_specs = [_full_spec(s) for s in
               ((IND, HH), (1, HH), (HH, HH), (1, HH), (HH, HH), (1, HH),
                (HH, 2), (1, 1))]

    table1, self1, sdst1 = pl.pallas_call(
        _tc1_body,
        grid=(NBLK,),
        in_specs=[_row_spec(TT * IND)] + w_specs,
        out_specs=[_row_spec(DD), _row_spec(160), _row_spec(16)],
        out_shape=[jax.ShapeDtypeStruct((NPAD, DD), _f32),
                   jax.ShapeDtypeStruct((NPAD, 160), _f32),
                   jax.ShapeDtypeStruct((NPAD, 16), _f32)],
    )(x2d, W_h, bh, g1_Wz, g1_bz.reshape(1, HH), g1_Wself,
      g1_bself.reshape(1, HH), Wa1, ba1)

    part1 = _sc_edge_pass(table1, sdst1, src, dst)

    part_spec = pl.BlockSpec((NCORES, BLK, DD), lambda i: (0, i, 0))
    table2, self2, sdst2 = pl.pallas_call(
        _tc2_body,
        grid=(NBLK,),
        in_specs=[part_spec, _row_spec(160), _row_spec(1)] + w_specs[2:],
        out_specs=[_row_spec(DD), _row_spec(160), _row_spec(16)],
        out_shape=[jax.ShapeDtypeStruct((NPAD, DD), _f32),
                   jax.ShapeDtypeStruct((NPAD, 160), _f32),
                   jax.ShapeDtypeStruct((NPAD, 16), _f32)],
    )(part1, self1, snorm, g2_Wz, g2_bz.reshape(1, HH), g2_Wself,
      g2_bself.reshape(1, HH), Wa2, ba2)

    part2 = _sc_edge_pass(table2, sdst2, src, dst)

    g3 = 3 * GHD
    gru_specs = [_full_spec(s) for s in
                 ((HH, g3), (GHD, g3), (1, g3), (1, g3),
                  (GHD, g3), (GHD, g3), (1, g3), (1, g3),
                  (2, g3), (GHD, g3), (1, g3), (1, g3),
                  (GHD, g3), (GHD, g3), (1, g3), (1, g3),
                  (GHD, 2), (1, 2))]

    out_p = pl.pallas_call(
        _tc3_body,
        grid=(NBLK,),
        in_specs=[part_spec, _row_spec(160), _row_spec(1),
                  _row_spec(TT * IND)] + gru_specs,
        out_specs=_row_spec(2 * NPRED),
        out_shape=jax.ShapeDtypeStruct((NPAD, 2 * NPRED), _f32),
    )(part2, self2, snorm, x2d,
      enc_Wih0, enc_Whh0, enc_bih0.reshape(1, g3), enc_bhh0.reshape(1, g3),
      enc_Wih1, enc_Whh1, enc_bih1.reshape(1, g3), enc_bhh1.reshape(1, g3),
      dec_Wih0, dec_Whh0, dec_bih0.reshape(1, g3), dec_bhh0.reshape(1, g3),
      dec_Wih1, dec_Whh1, dec_bih1.reshape(1, g3), dec_bhh1.reshape(1, g3),
      dec_Wout, dec_bout.reshape(1, 2))

    return out_p[:NN].reshape(NN, NPRED, 2)
